# baseline (speedup 1.0000x reference)
import jax
import jax.numpy as jnp
from jax.experimental import pallas as pl
from jax.experimental.pallas import tpu as pltpu

_N = 10000
_BLOCK = 10000


def _cell_body(x_ref, h_ref, wz_ref, wr_ref, wh_ref, bz_ref, br_ref, bh_ref,
               wlin_ref, blin_ref, out_ref, hnew_ref):
    d_in = x_ref.shape[1]
    wz = wz_ref[0, 0] + wz_ref[1, 0]
    wr = wr_ref[0, 0] + wr_ref[1, 0]
    wh = wh_ref[0, 0] + wh_ref[1, 0]
    x_b = x_ref[...]
    h_b = h_ref[...]
    z = jax.nn.sigmoid(
        jnp.dot(x_b, wz[:d_in], preferred_element_type=jnp.float32)
        + jnp.dot(h_b, wz[d_in:], preferred_element_type=jnp.float32)
        + bz_ref[...])
    r = jax.nn.sigmoid(
        jnp.dot(x_b, wr[:d_in], preferred_element_type=jnp.float32)
        + jnp.dot(h_b, wr[d_in:], preferred_element_type=jnp.float32)
        + br_ref[...])
    ht = jnp.tanh(
        jnp.dot(x_b, wh[:d_in], preferred_element_type=jnp.float32)
        + jnp.dot(h_b * r, wh[d_in:], preferred_element_type=jnp.float32)
        + bh_ref[...])
    h_new = z * h_b + (1.0 - z) * ht
    hnew_ref[...] = h_new
    out_ref[...] = (jnp.dot(jnp.maximum(h_new, 0.0), wlin_ref[...],
                            preferred_element_type=jnp.float32) + blin_ref[...])


def kernel(x, edge_index, edge_weight, h, Wz, bz, Wr, br, Wh, bh, W_lin, b_lin):
    del edge_index, edge_weight
    d_hid = h.shape[1]
    bz2, br2, bh2, blin2 = bz[None], br[None], bh[None], b_lin[None]

    grid = (_N // _BLOCK,)
    full = lambda a: pl.BlockSpec(a.shape, lambda i: (0,) * a.ndim)
    out, h_new = pl.pallas_call(
        _cell_body,
        grid=grid,
        in_specs=[
            pl.BlockSpec((_BLOCK, x.shape[1]), lambda i: (i, 0)),
            pl.BlockSpec((_BLOCK, d_hid), lambda i: (i, 0)),
            full(Wz), full(Wr), full(Wh),
            full(bz2), full(br2), full(bh2),
            full(W_lin), full(blin2),
        ],
        out_specs=[
            pl.BlockSpec((_BLOCK, W_lin.shape[1]), lambda i: (i, 0)),
            pl.BlockSpec((_BLOCK, d_hid), lambda i: (i, 0)),
        ],
        out_shape=[
            jax.ShapeDtypeStruct((_N, W_lin.shape[1]), jnp.float32),
            jax.ShapeDtypeStruct((_N, d_hid), jnp.float32),
        ],
        compiler_params=pltpu.CompilerParams(
            dimension_semantics=("parallel",),
        ),
    )(x, h, Wz, Wr, Wh, bz2, br2, bh2, W_lin, blin2)
    return (out, h_new)


# speedup vs baseline: 3.0673x; 3.0673x over previous
import jax
import jax.numpy as jnp
from jax import lax
from jax.experimental import pallas as pl
from jax.experimental.pallas import tpu as pltpu

_N = 10000

_DN_RT = (((1,), (1,)), ((), ()))


def _cell_body(x_ref, ht_ref, wzt_ref, wrt_ref, wht_ref, bz_ref, br_ref,
               bh_ref, wlt_ref, blt_ref, outt_ref, hnewt_ref):
    d_in = x_ref.shape[1]
    wz = wzt_ref[0, 0] + wzt_ref[1, 0]
    wr = wrt_ref[0, 0] + wrt_ref[1, 0]
    wh = wht_ref[0, 0] + wht_ref[1, 0]
    wzr = jnp.concatenate([wz, wr], axis=0)
    x_b = x_ref[...]
    ht_b = ht_ref[...]
    h_nat = jnp.transpose(ht_b)
    b_zr = jnp.concatenate([bz_ref[...], br_ref[...]], axis=1)
    g_zr = (lax.dot_general(x_b, wzr[:, :d_in], _DN_RT,
                            preferred_element_type=jnp.float32)
            + lax.dot_general(h_nat, wzr[:, d_in:], _DN_RT,
                              preferred_element_type=jnp.float32)
            + b_zr)
    zr = jax.nn.sigmoid(g_zr)
    z = zr[:, :32]
    r = zr[:, 32:]
    g_h = (lax.dot_general(x_b, wh[:, :d_in], _DN_RT,
                           preferred_element_type=jnp.float32)
           + lax.dot_general(h_nat * r, wh[:, d_in:], _DN_RT,
                             preferred_element_type=jnp.float32)
           + bh_ref[...])
    htl = jnp.tanh(g_h)
    h_new = z * h_nat + (1.0 - z) * htl
    hnewt_ref[...] = jnp.transpose(h_new)
    out = (lax.dot_general(jnp.maximum(h_new, 0.0), wlt_ref[...], _DN_RT,
                           preferred_element_type=jnp.float32)
           + blt_ref[...])
    outt_ref[...] = jnp.transpose(out)


def kernel(x, edge_index, edge_weight, h, Wz, bz, Wr, br, Wh, bh, W_lin, b_lin):
    del edge_index, edge_weight
    d_hid = h.shape[1]
    d_out = W_lin.shape[1]
    ht = h.T
    wzt = jnp.transpose(Wz, (0, 1, 3, 2))
    wrt = jnp.transpose(Wr, (0, 1, 3, 2))
    wht = jnp.transpose(Wh, (0, 1, 3, 2))
    wlt = W_lin.T
    bz2, br2, bh2, blt = bz[None], br[None], bh[None], b_lin[None]

    full = lambda a: pl.BlockSpec(a.shape, lambda: (0,) * a.ndim)
    out_t, h_new_t = pl.pallas_call(
        _cell_body,
        grid=(),
        in_specs=[full(x), full(ht), full(wzt), full(wrt), full(wht),
                  full(bz2), full(br2), full(bh2), full(wlt), full(blt)],
        out_specs=[
            pl.BlockSpec((d_out, _N), lambda: (0, 0)),
            pl.BlockSpec((d_hid, _N), lambda: (0, 0)),
        ],
        out_shape=[
            jax.ShapeDtypeStruct((d_out, _N), jnp.float32),
            jax.ShapeDtypeStruct((d_hid, _N), jnp.float32),
        ],
    )(x, ht, wzt, wrt, wht, bz2, br2, bh2, wlt, blt)
    return (out_t.T, h_new_t.T)
